# Initial kernel scaffold; baseline (speedup 1.0000x reference)
import jax
import jax.numpy as jnp
from jax.experimental import pallas as pl


def kernel(inputs, categories, mask_positions, tokens_embedding):
    raise NotImplementedError("write your pallas kernel here")


# speedup vs baseline: 1.4738x; 1.4738x over previous
import jax
import jax.numpy as jnp
from jax.experimental import pallas as pl
from jax.experimental.pallas import tpu as pltpu

B, S, D = 1024, 200, 128
GB = 8


def _body(pos_ref, x_ref, m_ref, o_ref):
    x = x_ref[...]
    m = m_ref[...]
    p = pos_ref[...].reshape(GB, 1, 1)
    row = jax.lax.broadcasted_iota(jnp.int32, (GB, S, D), 1)
    o_ref[...] = jnp.where(row == p, m[None, :, :], x)


def kernel(inputs, categories, mask_positions, tokens_embedding):
    del categories
    pos = mask_positions.astype(jnp.int32)
    grid = (B // GB,)
    out = pl.pallas_call(
        _body,
        grid=grid,
        in_specs=[
            pl.BlockSpec((GB, 1), lambda i: (i, 0)),
            pl.BlockSpec((GB, S, D), lambda i: (i, 0, 0)),
            pl.BlockSpec((1, D), lambda i: (0, 0)),
        ],
        out_specs=pl.BlockSpec((GB, S, D), lambda i: (i, 0, 0)),
        out_shape=jax.ShapeDtypeStruct((B, S, D), jnp.float32),
        compiler_params=pltpu.CompilerParams(
            dimension_semantics=("arbitrary",),
        ),
    )(pos, inputs, tokens_embedding)
    return out
